# baseline (speedup 1.0000x reference)
import functools

import jax
import jax.numpy as jnp
import numpy as np
from jax.experimental import pallas as pl
from jax.experimental.pallas import tpu as pltpu

NH_ = 16
DH_ = 128
K_ = 128
F32 = jnp.float32
BF16 = jnp.bfloat16


def _qkv_kernel(x_ref, wq_ref, wk_ref, wv_ref, q_ref, k_ref, v_ref):
    x = x_ref[...]
    q_ref[...] = jnp.dot(x, wq_ref[...], preferred_element_type=F32).astype(BF16)
    k_ref[...] = jnp.dot(x, wk_ref[...], preferred_element_type=F32).astype(BF16)
    v_ref[...] = jnp.dot(x, wv_ref[...], preferred_element_type=F32).astype(BF16)


def _attn_kernel(q_ref, k_ref, v_ref, w1t_ref, b1_ref, w2_ref, b2_ref, u_ref,
                 o_ref, *, kk):
    S = q_ref.shape[2]
    k_bh = k_ref[0, 0]
    h1 = jnp.dot(k_bh, w1t_ref[...], preferred_element_type=F32)
    h1 = jnp.maximum(h1 + b1_ref[...], 0.0).astype(BF16)
    dn_t = (((1,), (1,)), ((), ()))
    s = jax.lax.dot_general(w2_ref[...], h1, dn_t,
                            preferred_element_type=F32) + b2_ref[...]

    def tau_body(_, lohi):
        lo, hi = lohi
        mid = 0.5 * (lo + hi)
        mass = jnp.sum(jnp.maximum(s - mid, 0.0))
        return jnp.where(mass >= float(kk), mid, lo), jnp.where(mass >= float(kk), hi, mid)

    lo0 = jnp.min(s) - 1.0
    hi0 = jnp.max(s)
    lo, hi = jax.lax.fori_loop(0, 48, tau_body, (lo0, hi0))
    tau = 0.5 * (lo + hi)
    sel = jnp.maximum(s - tau, 0.0)

    def bnd_body(_, lohi):
        lo, hi = lohi
        mid = 0.5 * (lo + hi)
        cnt = jnp.sum(jnp.where(sel > mid, 1.0, 0.0))
        return jnp.where(cnt >= float(kk), mid, lo), jnp.where(cnt >= float(kk), hi, mid)

    blo, bhi = jax.lax.fori_loop(0, 50, bnd_body, (jnp.float32(-1.0), jnp.max(sel)))
    strict = sel > bhi
    n_strict = jnp.sum(jnp.where(strict, 1.0, 0.0))
    need = float(kk) - n_strict
    cand = jnp.logical_and(sel > blo, jnp.logical_not(strict))
    cand_b = jnp.where(cand, 1.0, 0.0).astype(BF16)
    rank_c = jnp.dot(cand_b, u_ref[...], preferred_element_type=F32)
    fill = jnp.logical_and(cand, rank_c <= need)
    mask = jnp.logical_or(strict, fill)
    mask_b = jnp.where(mask, 1.0, 0.0).astype(BF16)
    rank = jnp.dot(mask_b, u_ref[...], preferred_element_type=F32)
    slot = jnp.where(mask, rank - 1.0, -1.0)

    iota = jax.lax.broadcasted_iota(jnp.int32, (kk, S), 0)
    P = jnp.where(iota == slot.astype(jnp.int32), 1.0, 0.0).astype(BF16)
    k_sel = jnp.dot(P, k_bh, preferred_element_type=F32).astype(BF16)
    v_sel = jnp.dot(P, v_ref[0, 0], preferred_element_type=F32).astype(BF16)

    att = jax.lax.dot_general(q_ref[0, 0], k_sel, dn_t,
                              preferred_element_type=F32)
    att = att * (1.0 / np.sqrt(DH_))
    m = jnp.max(att, axis=1, keepdims=True)
    p = jnp.exp(att - m)
    a = (p / jnp.sum(p, axis=1, keepdims=True)).astype(BF16)
    o_ref[0, 0] = jnp.dot(a, v_sel, preferred_element_type=F32).astype(BF16)


def _proj_kernel(x_ref, w_ref, o_ref):
    o_ref[...] = jnp.dot(x_ref[...], w_ref[...], preferred_element_type=F32)


def kernel(x, Wq, Wk, Wv, Wo, W1, b1, W2, b2):
    b, S, hid = x.shape
    M = b * S
    MB = 512
    x2 = x.reshape(M, hid).astype(BF16)

    q2, k2, v2 = pl.pallas_call(
        _qkv_kernel,
        grid=(M // MB,),
        in_specs=[
            pl.BlockSpec((MB, hid), lambda i: (i, 0)),
            pl.BlockSpec((hid, hid), lambda i: (0, 0)),
            pl.BlockSpec((hid, hid), lambda i: (0, 0)),
            pl.BlockSpec((hid, hid), lambda i: (0, 0)),
        ],
        out_specs=[pl.BlockSpec((MB, hid), lambda i: (i, 0))] * 3,
        out_shape=[jax.ShapeDtypeStruct((M, hid), BF16)] * 3,
    )(x2, Wq.T.astype(BF16), Wk.T.astype(BF16), Wv.T.astype(BF16))

    def heads(t):
        return t.reshape(b, S, NH_, DH_).transpose(0, 2, 1, 3)

    q4, k4, v4 = heads(q2), heads(k2), heads(v2)

    ii = jnp.arange(S, dtype=jnp.int32)
    U = (ii[:, None] <= ii[None, :]).astype(BF16)

    bh_spec = pl.BlockSpec((1, 1, S, DH_), lambda i, j: (i, j, 0, 0))
    cst = lambda shape: pl.BlockSpec(shape, lambda i, j: tuple(0 for _ in shape))
    attn_out = pl.pallas_call(
        functools.partial(_attn_kernel, kk=min(K_, S)),
        grid=(b, NH_),
        in_specs=[
            bh_spec, bh_spec, bh_spec,
            cst((DH_, DH_)), cst((1, DH_)), cst((1, DH_)), cst((1, 1)),
            cst((S, S)),
        ],
        out_specs=bh_spec,
        out_shape=jax.ShapeDtypeStruct((b, NH_, S, DH_), BF16),
    )(q4, k4, v4,
      W1.T.astype(BF16), b1.reshape(1, DH_), W2.astype(BF16),
      b2.reshape(1, 1), U)

    ao2 = attn_out.transpose(0, 2, 1, 3).reshape(M, hid)

    out = pl.pallas_call(
        _proj_kernel,
        grid=(M // MB,),
        in_specs=[
            pl.BlockSpec((MB, hid), lambda i: (i, 0)),
            pl.BlockSpec((hid, hid), lambda i: (0, 0)),
        ],
        out_specs=pl.BlockSpec((MB, hid), lambda i: (i, 0)),
        out_shape=jax.ShapeDtypeStruct((M, hid), F32),
    )(ao2, Wo.T.astype(BF16))
    return out.reshape(b, S, hid)


# speedup vs baseline: 2.1029x; 2.1029x over previous
import functools

import jax
import jax.numpy as jnp
import numpy as np
from jax.experimental import pallas as pl
from jax.experimental.pallas import tpu as pltpu

NH_ = 16
DH_ = 128
K_ = 128
F32 = jnp.float32
BF16 = jnp.bfloat16
DN_T = (((1,), (1,)), ((), ()))


def _qkv_kernel(x_ref, wq_ref, wk_ref, wv_ref, q_ref, k_ref, v_ref):
    x = x_ref[...]
    q_ref[...] = jnp.dot(x, wq_ref[...], preferred_element_type=F32).astype(BF16)
    k_ref[...] = jnp.dot(x, wk_ref[...], preferred_element_type=F32).astype(BF16)
    v_ref[...] = jnp.dot(x, wv_ref[...], preferred_element_type=F32).astype(BF16)


def _select_kernel(k_ref, w1t_ref, b1_ref, w2_ref, b2_ref, u_ref,
                   slot_ref, *, kk):
    nb, nh = k_ref.shape[0], k_ref.shape[1]
    rows = []
    for bi in range(nb):
        for hi in range(nh):
            h1 = jnp.dot(k_ref[bi, hi], w1t_ref[...], preferred_element_type=F32)
            h1 = jnp.maximum(h1 + b1_ref[...], 0.0).astype(BF16)
            rows.append(jax.lax.dot_general(w2_ref[...], h1, DN_T,
                                            preferred_element_type=F32))
    s = jnp.concatenate(rows, axis=0) + b2_ref[...]

    def tau_body(_, lohi):
        lo, hi = lohi
        mid = 0.5 * (lo + hi)
        mass = jnp.sum(jnp.maximum(s - mid, 0.0), axis=1, keepdims=True)
        go = mass >= float(kk)
        return jnp.where(go, mid, lo), jnp.where(go, hi, mid)

    lo0 = jnp.min(s, axis=1, keepdims=True) - 1.0
    hi0 = jnp.max(s, axis=1, keepdims=True)
    lo, hi = jax.lax.fori_loop(0, 44, tau_body, (lo0, hi0))
    sel = jnp.maximum(s - 0.5 * (lo + hi), 0.0)

    def bnd_body(_, lohi):
        lo, hi = lohi
        mid = 0.5 * (lo + hi)
        cnt = jnp.sum(jnp.where(sel > mid, 1.0, 0.0), axis=1, keepdims=True)
        go = cnt >= float(kk)
        return jnp.where(go, mid, lo), jnp.where(go, hi, mid)

    blo0 = jnp.full_like(lo0, -1.0)
    bhi0 = jnp.max(sel, axis=1, keepdims=True)
    blo, bhi = jax.lax.fori_loop(0, 50, bnd_body, (blo0, bhi0))
    strict = sel > bhi
    need = float(kk) - jnp.sum(jnp.where(strict, 1.0, 0.0), axis=1, keepdims=True)
    cand = jnp.logical_and(sel > blo, jnp.logical_not(strict))
    cand_b = jnp.where(cand, 1.0, 0.0).astype(BF16)
    rank_c = jnp.dot(cand_b, u_ref[...], preferred_element_type=F32)
    fill = jnp.logical_and(cand, rank_c <= need)
    mask = jnp.logical_or(strict, fill)
    mask_b = jnp.where(mask, 1.0, 0.0).astype(BF16)
    rank = jnp.dot(mask_b, u_ref[...], preferred_element_type=F32)
    slot_ref[...] = jnp.where(mask, rank - 1.0, -1.0)


def _attn_kernel(q_ref, k_ref, v_ref, slot_ref, o_ref, *, kk):
    S = q_ref.shape[2]
    slot = slot_ref[0, 0].astype(jnp.int32)
    iota = jax.lax.broadcasted_iota(jnp.int32, (kk, S), 0)
    P = jnp.where(iota == slot, 1.0, 0.0).astype(BF16)
    k_sel = jnp.dot(P, k_ref[0, 0], preferred_element_type=F32).astype(BF16)
    v_sel = jnp.dot(P, v_ref[0, 0], preferred_element_type=F32).astype(BF16)
    att = jax.lax.dot_general(q_ref[0, 0], k_sel, DN_T,
                              preferred_element_type=F32) * (1.0 / np.sqrt(DH_))
    m = jnp.max(att, axis=1, keepdims=True)
    p = jnp.exp(att - m)
    a = (p / jnp.sum(p, axis=1, keepdims=True)).astype(BF16)
    o_ref[0, 0] = jnp.dot(a, v_sel, preferred_element_type=F32).astype(BF16)


def _proj_kernel(x_ref, w_ref, o_ref):
    o_ref[...] = jnp.dot(x_ref[...], w_ref[...], preferred_element_type=F32)


def kernel(x, Wq, Wk, Wv, Wo, W1, b1, W2, b2):
    b, S, hid = x.shape
    M = b * S
    MB = 512
    kk = min(K_, S)
    x2 = x.reshape(M, hid).astype(BF16)

    q2, k2, v2 = pl.pallas_call(
        _qkv_kernel,
        grid=(M // MB,),
        in_specs=[
            pl.BlockSpec((MB, hid), lambda i: (i, 0)),
            pl.BlockSpec((hid, hid), lambda i: (0, 0)),
            pl.BlockSpec((hid, hid), lambda i: (0, 0)),
            pl.BlockSpec((hid, hid), lambda i: (0, 0)),
        ],
        out_specs=[pl.BlockSpec((MB, hid), lambda i: (i, 0))] * 3,
        out_shape=[jax.ShapeDtypeStruct((M, hid), BF16)] * 3,
    )(x2, Wq.T.astype(BF16), Wk.T.astype(BF16), Wv.T.astype(BF16))

    def heads(t):
        return t.reshape(b, S, NH_, DH_).transpose(0, 2, 1, 3)

    q4, k4, v4 = heads(q2), heads(k2), heads(v2)

    ii = jnp.arange(S, dtype=jnp.int32)
    U = (ii[:, None] <= ii[None, :]).astype(BF16)

    cB = lambda shape: pl.BlockSpec(shape, lambda i: tuple(0 for _ in shape))
    slot = pl.pallas_call(
        functools.partial(_select_kernel, kk=kk),
        grid=(1,),
        in_specs=[cB((b, NH_, S, DH_)),
                  cB((DH_, DH_)), cB((1, DH_)), cB((1, DH_)), cB((1, 1)),
                  cB((S, S))],
        out_specs=pl.BlockSpec((b * NH_, S), lambda i: (0, 0)),
        out_shape=jax.ShapeDtypeStruct((b * NH_, S), F32),
    )(k4, W1.T.astype(BF16), b1.reshape(1, DH_), W2.astype(BF16),
      b2.reshape(1, 1), U)
    slot4 = slot.reshape(b, NH_, 1, S)

    bh_spec = pl.BlockSpec((1, 1, S, DH_), lambda i, j: (i, j, 0, 0))
    attn_out = pl.pallas_call(
        functools.partial(_attn_kernel, kk=kk),
        grid=(b, NH_),
        in_specs=[bh_spec, bh_spec, bh_spec,
                  pl.BlockSpec((1, 1, 1, S), lambda i, j: (i, j, 0, 0))],
        out_specs=bh_spec,
        out_shape=jax.ShapeDtypeStruct((b, NH_, S, DH_), BF16),
    )(q4, k4, v4, slot4)

    ao2 = attn_out.transpose(0, 2, 1, 3).reshape(M, hid)

    out = pl.pallas_call(
        _proj_kernel,
        grid=(M // MB,),
        in_specs=[pl.BlockSpec((MB, hid), lambda i: (i, 0)),
                  pl.BlockSpec((hid, hid), lambda i: (0, 0))],
        out_specs=pl.BlockSpec((MB, hid), lambda i: (i, 0)),
        out_shape=jax.ShapeDtypeStruct((M, hid), F32),
    )(ao2, Wo.T.astype(BF16))
    return out.reshape(b, S, hid)


# speedup vs baseline: 3.2348x; 1.5383x over previous
import functools

import jax
import jax.numpy as jnp
import numpy as np
from jax.experimental import pallas as pl
from jax.experimental.pallas import tpu as pltpu

NH_ = 16
DH_ = 128
K_ = 128
F32 = jnp.float32
BF16 = jnp.bfloat16
DN_T = (((1,), (1,)), ((), ()))


def _qkv_kernel(x_ref, wq_ref, wk_ref, wv_ref, q_ref, k_ref, v_ref):
    x = x_ref[...].astype(BF16)
    q_ref[...] = jax.lax.dot_general(
        x, wq_ref[...], DN_T, preferred_element_type=F32).astype(BF16)
    k_ref[...] = jax.lax.dot_general(
        x, wk_ref[...], DN_T, preferred_element_type=F32).astype(BF16)
    v_ref[...] = jax.lax.dot_general(
        x, wv_ref[...], DN_T, preferred_element_type=F32).astype(BF16)


def _select_kernel(k_ref, w1_ref, b1_ref, w2_ref, b2_ref, u_ref,
                   slot_ref, *, kk, nb, nh):
    S = k_ref.shape[0] // nb
    rows = []
    for bi in range(nb):
        for hi in range(nh):
            kbh = k_ref[bi * S:(bi + 1) * S, hi * DH_:(hi + 1) * DH_]
            h1 = jax.lax.dot_general(kbh, w1_ref[...], DN_T,
                                     preferred_element_type=F32)
            h1 = jnp.maximum(h1 + b1_ref[...], 0.0).astype(BF16)
            rows.append(jax.lax.dot_general(w2_ref[...], h1, DN_T,
                                            preferred_element_type=F32))
    s = jnp.concatenate(rows, axis=0) + b2_ref[...]

    def tau_body(_, lohi):
        lo, hi = lohi
        mid = 0.5 * (lo + hi)
        mass = jnp.sum(jnp.maximum(s - mid, 0.0), axis=1, keepdims=True)
        go = mass >= float(kk)
        return jnp.where(go, mid, lo), jnp.where(go, hi, mid)

    lo0 = jnp.min(s, axis=1, keepdims=True) - 1.0
    hi0 = jnp.max(s, axis=1, keepdims=True)
    lo, hi = jax.lax.fori_loop(0, 44, tau_body, (lo0, hi0))
    sel = jnp.maximum(s - 0.5 * (lo + hi), 0.0)

    def bnd_body(_, lohi):
        lo, hi = lohi
        mid = 0.5 * (lo + hi)
        cnt = jnp.sum(jnp.where(sel > mid, 1.0, 0.0), axis=1, keepdims=True)
        go = cnt >= float(kk)
        return jnp.where(go, mid, lo), jnp.where(go, hi, mid)

    blo0 = jnp.full_like(lo0, -1.0)
    bhi0 = jnp.max(sel, axis=1, keepdims=True)
    blo, bhi = jax.lax.fori_loop(0, 50, bnd_body, (blo0, bhi0))
    strict = sel > bhi
    need = float(kk) - jnp.sum(jnp.where(strict, 1.0, 0.0), axis=1, keepdims=True)
    cand = jnp.logical_and(sel > blo, jnp.logical_not(strict))
    cand_b = jnp.where(cand, 1.0, 0.0).astype(BF16)
    rank_c = jnp.dot(cand_b, u_ref[...], preferred_element_type=F32)
    fill = jnp.logical_and(cand, rank_c <= need)
    mask = jnp.logical_or(strict, fill)
    mask_b = jnp.where(mask, 1.0, 0.0).astype(BF16)
    rank = jnp.dot(mask_b, u_ref[...], preferred_element_type=F32)
    slot_ref[...] = jnp.where(mask, rank - 1.0, -1.0)


def _attn_kernel(q_ref, k_ref, v_ref, slot_ref, o_ref, *, kk):
    S = q_ref.shape[0]
    slot = slot_ref[0].astype(jnp.int32)
    iota = jax.lax.broadcasted_iota(jnp.int32, (kk, S), 0)
    P = jnp.where(iota == slot, 1.0, 0.0).astype(BF16)
    k_sel = jnp.dot(P, k_ref[...], preferred_element_type=F32).astype(BF16)
    v_sel = jnp.dot(P, v_ref[...], preferred_element_type=F32).astype(BF16)
    att = jax.lax.dot_general(q_ref[...], k_sel, DN_T,
                              preferred_element_type=F32) * (1.0 / np.sqrt(DH_))
    m = jnp.max(att, axis=1, keepdims=True)
    p = jnp.exp(att - m)
    a = (p / jnp.sum(p, axis=1, keepdims=True)).astype(BF16)
    o_ref[...] = jnp.dot(a, v_sel, preferred_element_type=F32).astype(BF16)


def _proj_kernel(x_ref, w_ref, o_ref):
    o_ref[...] = jax.lax.dot_general(x_ref[...], w_ref[...], DN_T,
                                     preferred_element_type=F32)


def kernel(x, Wq, Wk, Wv, Wo, W1, b1, W2, b2):
    b, S, hid = x.shape
    M = b * S
    MB = 512
    kk = min(K_, S)
    R = b * NH_
    x2 = x.reshape(M, hid)

    row_spec = pl.BlockSpec((MB, hid), lambda i: (i, 0))
    w_spec = pl.BlockSpec((hid, hid), lambda i: (0, 0))
    q2, k2, v2 = pl.pallas_call(
        _qkv_kernel,
        grid=(M // MB,),
        in_specs=[row_spec, w_spec, w_spec, w_spec],
        out_specs=[row_spec] * 3,
        out_shape=[jax.ShapeDtypeStruct((M, hid), BF16)] * 3,
    )(x2, Wq.astype(BF16), Wk.astype(BF16), Wv.astype(BF16))

    ii = jnp.arange(S, dtype=jnp.int32)
    U = (ii[:, None] <= ii[None, :]).astype(BF16)

    cB = lambda shape: pl.BlockSpec(shape, lambda i: tuple(0 for _ in shape))
    slot = pl.pallas_call(
        functools.partial(_select_kernel, kk=kk, nb=b, nh=NH_),
        grid=(1,),
        in_specs=[cB((M, hid)),
                  cB((DH_, DH_)), cB((1, DH_)), cB((1, DH_)), cB((1, 1)),
                  cB((S, S))],
        out_specs=pl.BlockSpec((R, S), lambda i: (0, 0)),
        out_shape=jax.ShapeDtypeStruct((R, S), F32),
    )(k2, W1.astype(BF16), b1.reshape(1, DH_), W2.astype(BF16),
      b2.reshape(1, 1), U)
    slot3 = slot.reshape(R, 1, S)

    bh_spec = pl.BlockSpec((S, DH_), lambda i, j: (i, j))
    attn_out = pl.pallas_call(
        functools.partial(_attn_kernel, kk=kk),
        grid=(b, NH_),
        in_specs=[bh_spec, bh_spec, bh_spec,
                  pl.BlockSpec((1, 1, S), lambda i, j: (i * NH_ + j, 0, 0))],
        out_specs=bh_spec,
        out_shape=jax.ShapeDtypeStruct((M, hid), BF16),
    )(q2, k2, v2, slot3)

    out = pl.pallas_call(
        _proj_kernel,
        grid=(M // MB,),
        in_specs=[row_spec, w_spec],
        out_specs=row_spec,
        out_shape=jax.ShapeDtypeStruct((M, hid), F32),
    )(attn_out, Wo.astype(BF16))
    return out.reshape(b, S, hid)
